# baseline (speedup 1.0000x reference)
import math

import jax
import jax.numpy as jnp
import numpy as np
from jax.experimental import pallas as pl
from jax.experimental.pallas import tpu as pltpu

_D = 1024
_H = 16
_E = _D // _H
_L = 2048
_C_FACTOR = 5
_SAMPLE_K = _C_FACTOR * int(math.ceil(math.log(_L)))
_N_TOP = _C_FACTOR * int(math.ceil(math.log(_L)))
_NEG = float(-(2 ** 32) + 1)
_TM = 512


def _threefry2x32(k1, k2, x0, x1):
    rot = ([13, 15, 26, 6], [17, 29, 16, 24])
    u32 = np.uint32
    ks = [u32(k1), u32(k2), u32(u32(k1) ^ u32(k2) ^ u32(0x1BD11BDA))]
    x0 = (x0 + ks[0]).astype(np.uint32)
    x1 = (x1 + ks[1]).astype(np.uint32)
    for i in range(5):
        for r in rot[i % 2]:
            x0 = (x0 + x1).astype(np.uint32)
            x1 = ((x1 << u32(r)) | (x1 >> u32(32 - r))).astype(np.uint32)
            x1 = x0 ^ x1
        x0 = (x0 + ks[(i + 1) % 3]).astype(np.uint32)
        x1 = (x1 + ks[(i + 2) % 3] + u32(i + 1)).astype(np.uint32)
    return x0, x1


def _sample_indices():
    b1, b2 = _threefry2x32(np.uint32(0), np.uint32(42),
                           np.zeros(2, np.uint32),
                           np.arange(2, dtype=np.uint32))
    n = _L * _SAMPLE_K
    counts_lo = np.arange(n, dtype=np.uint32)
    counts_hi = np.zeros(n, np.uint32)
    lo0, lo1 = _threefry2x32(b1[1], b2[1], counts_hi, counts_lo)
    lower_bits = lo0 ^ lo1
    return (lower_bits % np.uint32(_L)).astype(np.int32).reshape(_L, _SAMPLE_K)


def _sample_count_matrix_t():
    idx = _sample_indices()
    c = np.zeros((_L, _L), np.float32)
    np.add.at(c, (np.arange(_L)[:, None], idx), 1.0)
    return np.ascontiguousarray(c.T)


_CT_NP = _sample_count_matrix_t()
_CM_NP = np.where(_CT_NP > 0.0, np.float32(0.0),
                  np.float32(-1e30)).astype(np.float32)


def _proj3_body(q_ref, k_ref, v_ref, wq_ref, bq_ref, wk_ref, bk_ref,
                wv_ref, bv_ref, qo_ref, ko_ref, vo_ref):
    qo_ref[...] = (
        jnp.dot(q_ref[...], wq_ref[...], preferred_element_type=jnp.float32)
        + bq_ref[...]
    )
    ko_ref[...] = (
        jnp.dot(k_ref[...], wk_ref[...], preferred_element_type=jnp.float32)
        + bk_ref[...]
    )
    vo_ref[...] = (
        jnp.dot(v_ref[...], wv_ref[...], preferred_element_type=jnp.float32)
        + bv_ref[...]
    )


def _head_body(qh_ref, kh_ref, vh_ref, ct_ref, cm_ref, o_ref, qr_scr):
    kh = kh_ref[0]
    qh = qh_ref[0]
    khb = kh.astype(jnp.bfloat16)

    p = jax.lax.dot_general(
        ct_ref[...], khb, (((0,), (0,)), ((), ())),
        preferred_element_type=jnp.float32,
    )
    qt_t = jnp.transpose(qh)
    p_t = jnp.transpose(p)
    ssum = jnp.sum(qt_t * p_t, axis=0, keepdims=True)

    m_rows = []
    for t in range(_L // 256):
        sl = slice(t * 256, (t + 1) * 256)
        st = jax.lax.dot_general(
            kh, qh[sl, :], (((1,), (1,)), ((), ())),
            preferred_element_type=jnp.float32,
        )
        smax = jnp.max(st + cm_ref[:, sl], axis=0, keepdims=True)
        m_rows.append(smax - ssum[:, sl] * (1.0 / _L))
    m = jnp.concatenate(m_rows, axis=0)

    qidx = (
        jax.lax.broadcasted_iota(jnp.int32, (8, 256), 0) * 256
        + jax.lax.broadcasted_iota(jnp.int32, (8, 256), 1)
    )
    qr_scr[...] = jnp.zeros((64, _E), jnp.float32)
    th = jnp.full((64, 1), jnp.int32(_L))
    row64 = jax.lax.broadcasted_iota(jnp.int32, (64, 1), 0)
    idxs = []
    for j in range(_N_TOP):
        mx = jnp.max(m)
        ii = jnp.min(jnp.where(m == mx, qidx, jnp.int32(2 ** 30)))
        idxs.append(ii)
        m = jnp.where(qidx == ii, jnp.float32(-1e30), m)
        th = jnp.where(row64 == j, ii, th)
        qr_scr[j:j + 1, :] = qh_ref[0, pl.ds(ii, 1), :]

    scores = jax.lax.dot_general(
        qr_scr[...], kh, (((1,), (1,)), ((), ())),
        preferred_element_type=jnp.float32,
    ) * (1.0 / math.sqrt(_E))
    kcol = jax.lax.broadcasted_iota(jnp.int32, (64, _L), 1)
    scores = jnp.where(kcol <= th, scores, jnp.float32(_NEG))
    smax2 = jnp.max(scores, axis=1, keepdims=True)
    pr = jnp.exp(scores - smax2)
    pr = pr / jnp.sum(pr, axis=1, keepdims=True)
    out_sel = jnp.dot(pr, vh_ref[0], preferred_element_type=jnp.float32)

    r256 = jax.lax.broadcasted_iota(jnp.int32, (256, 256), 0)
    c256 = jax.lax.broadcasted_iota(jnp.int32, (256, 256), 1)
    tril = jnp.where(r256 >= c256, jnp.float32(1.0), jnp.float32(0.0))
    carry = jnp.zeros((1, _E), jnp.float32)
    for t in range(_L // 256):
        vt = vh_ref[0, t * 256:(t + 1) * 256, :]
        o_ref[0, t * 256:(t + 1) * 256, :] = (
            jnp.dot(tril, vt, preferred_element_type=jnp.float32) + carry
        )
        carry = carry + jnp.sum(vt, axis=0, keepdims=True)

    for j in range(_N_TOP):
        o_ref[0, pl.ds(idxs[j], 1), :] = out_sel[j:j + 1, :]


def kernel(q, k, v, mask, Wq, bq, Wk, bk, Wv, bv):
    B, L, D = q.shape
    rows = B * L
    q2 = q.reshape(rows, D)
    k2 = k.reshape(rows, D)
    v2 = v.reshape(rows, D)

    mat_spec = pl.BlockSpec((D, D), lambda g: (0, 0))
    bias_spec = pl.BlockSpec((1, D), lambda g: (0, 0))
    row_spec = pl.BlockSpec((_TM, D), lambda g: (g, 0))
    qp, kp, vp = pl.pallas_call(
        _proj3_body,
        grid=(rows // _TM,),
        in_specs=[row_spec, row_spec, row_spec,
                  mat_spec, bias_spec, mat_spec, bias_spec,
                  mat_spec, bias_spec],
        out_specs=[row_spec, row_spec, row_spec],
        out_shape=[jax.ShapeDtypeStruct((rows, D), jnp.float32)] * 3,
    )(q2, k2, v2, Wq, bq.reshape(1, D), Wk, bk.reshape(1, D),
      Wv, bv.reshape(1, D))

    bh = B * _H
    qh = qp.reshape(bh, _L, _E)
    kh = kp.reshape(bh, _L, _E)
    vh = vp.reshape(bh, _L, _E)
    ct = jnp.asarray(_CT_NP.astype(jnp.bfloat16))
    cm = jnp.asarray(_CM_NP)

    head_spec = pl.BlockSpec((1, _L, _E), lambda g: (g, 0, 0))
    const_spec = pl.BlockSpec((_L, _L), lambda g: (0, 0))
    ctx = pl.pallas_call(
        _head_body,
        grid=(bh,),
        in_specs=[head_spec, head_spec, head_spec, const_spec, const_spec],
        out_specs=head_spec,
        out_shape=jax.ShapeDtypeStruct((bh, _L, _E), jnp.float32),
        scratch_shapes=[pltpu.VMEM((64, _E), jnp.float32)],
    )(qh, kh, vh, ct, cm)

    return ctx.reshape(B, L, D)
